# baseline (speedup 1.0000x reference)
import functools

import jax
import jax.numpy as jnp
from jax import lax
from jax.experimental import pallas as pl
from jax.experimental.pallas import tpu as pltpu
from jax.experimental.pallas import tpu_sc as plsc

NN = 100000
EE = 6400000
LANES = 128
ROWS = EE // LANES
NP = 100352
NR = NP // LANES
NC, NS = 2, 16
NW = NC * NS
SL = NP // NS

_MESH = plsc.VectorSubcoreMesh(
    core_axis_name="c", subcore_axis_name="s", num_cores=NC, num_subcores=NS)


def _sc_degree_body(er, z, out, dst_v, ones_v, acc_sh, lsem, ssem):
    K = 16
    chunks = ROWS // K
    rem = chunks % NW
    cid = lax.axis_index("c")
    sid = lax.axis_index("s")
    wid = sid * NC + cid

    @pl.when(sid == 0)
    def _():
        pltpu.sync_copy(z, acc_sh)
    for i in range(8):
        ones_v[pl.ds(16 * i, 16)] = jnp.ones((16,), jnp.float32)
    plsc.subcore_barrier()

    n_i = jnp.where(wid < rem, chunks // NW + 1, chunks // NW)
    pltpu.async_copy(er.at[1, pl.ds(wid * K, K)], dst_v.at[0], lsem)

    def body(i, carry):
        b = lax.rem(i, 2)
        pltpu.make_async_copy(er.at[1, pl.ds(0, K)], dst_v.at[b],
                              lsem).wait()

        @pl.when(i + 1 < n_i)
        def _():
            qn = wid + (i + 1) * NW
            pltpu.async_copy(er.at[1, pl.ds(qn * K, K)], dst_v.at[1 - b],
                             lsem)
        ds = [pltpu.async_copy(ones_v, acc_sh.at[dst_v.at[b, j]], ssem,
                               add=True)
              for j in range(K)]
        for d in ds:
            d.wait()
        return carry

    lax.fori_loop(0, n_i, body, 0)
    plsc.subcore_barrier()
    pltpu.sync_copy(acc_sh.at[pl.ds(sid * SL, SL)],
                    out.at[cid, pl.ds(sid * SL, SL)])


_sc_degree = functools.partial(
    pl.kernel,
    out_type=jax.ShapeDtypeStruct((NC, NP), jnp.float32),
    mesh=_MESH,
    scratch_types=[
        pltpu.VMEM((2, 16, LANES), jnp.int32),
        pltpu.VMEM((LANES,), jnp.float32),
        pltpu.VMEM_SHARED((NP,), jnp.float32),
        pltpu.SemaphoreType.DMA,
        pltpu.SemaphoreType.DMA,
    ],
)(_sc_degree_body)


def _make_sc_conv(nch, K, hbm_gather=False):
    chunks = ROWS // K
    rem = chunks % NW

    def body(er, *refs):
        tabs_hbm = refs[:nch]
        z = refs[nch]
        out = refs[nch + 1]
        src_v, dst_v = refs[nch + 2], refs[nch + 3]
        vals = refs[nch + 4:nch + 4 + nch]
        tabs_sh = refs[nch + 4 + nch:nch + 4 + 2 * nch]
        accs_sh = refs[nch + 4 + 2 * nch:nch + 4 + 3 * nch]
        lsem, gsem, ssem = refs[nch + 4 + 3 * nch:nch + 7 + 3 * nch]

        cid = lax.axis_index("c")
        sid = lax.axis_index("s")
        wid = sid * NC + cid

        @pl.when(sid == 0)
        def _():
            for ch in range(nch):
                if not hbm_gather:
                    pltpu.sync_copy(tabs_hbm[ch], tabs_sh[ch])
                pltpu.sync_copy(z, accs_sh[ch])
        plsc.subcore_barrier()
        gsrc = tabs_hbm if hbm_gather else tabs_sh

        n_i = jnp.where(wid < rem, chunks // NW + 1, chunks // NW)
        pltpu.async_copy(er.at[0, pl.ds(wid * K, K)], src_v.at[0], lsem)
        pltpu.async_copy(er.at[1, pl.ds(wid * K, K)], dst_v.at[0], lsem)

        def loop(i, carry):
            b = lax.rem(i, 2)
            pltpu.make_async_copy(er.at[0, pl.ds(0, K)], src_v.at[b],
                                  lsem).wait()
            pltpu.make_async_copy(er.at[1, pl.ds(0, K)], dst_v.at[b],
                                  lsem).wait()

            @pl.when(i + 1 < n_i)
            def _():
                qn = wid + (i + 1) * NW
                pltpu.async_copy(er.at[0, pl.ds(qn * K, K)],
                                 src_v.at[1 - b], lsem)
                pltpu.async_copy(er.at[1, pl.ds(qn * K, K)],
                                 dst_v.at[1 - b], lsem)

            gds = [pltpu.async_copy(gsrc[ch].at[src_v.at[b, j]],
                                    vals[ch].at[j], gsem)
                   for j in range(K) for ch in range(nch)]
            sds = []
            for j in range(K):
                for ch in range(nch):
                    gds[j * nch + ch].wait()
                for ch in range(nch):
                    sds.append(pltpu.async_copy(
                        vals[ch].at[j], accs_sh[ch].at[dst_v.at[b, j]],
                        ssem, add=True))
            for d in sds:
                d.wait()
            return carry

        lax.fori_loop(0, n_i, loop, 0)
        plsc.subcore_barrier()
        for ch in range(nch):
            pltpu.sync_copy(accs_sh[ch].at[pl.ds(sid * SL, SL)],
                            out.at[cid, ch, pl.ds(sid * SL, SL)])

    scratch = [pltpu.VMEM((2, K, LANES), jnp.int32),
               pltpu.VMEM((2, K, LANES), jnp.int32)]
    scratch += [pltpu.VMEM((K, LANES), jnp.float32) for _ in range(nch)]
    scratch += [pltpu.VMEM_SHARED((NP,), jnp.float32) for _ in range(2 * nch)]
    scratch += [pltpu.SemaphoreType.DMA, pltpu.SemaphoreType.DMA,
                pltpu.SemaphoreType.DMA]
    return functools.partial(
        pl.kernel,
        out_type=jax.ShapeDtypeStruct((NC, nch, NP), jnp.float32),
        mesh=_MESH,
        scratch_types=scratch,
    )(body)


_sc_conv2ch = _make_sc_conv(2, 8, hbm_gather=True)
_sc_conv1ch = _make_sc_conv(1, 16, hbm_gather=True)


def _prep1_body(degp_ref, ft_ref, w_ref, dinv_ref, t0_ref, t1_ref):
    deg = degp_ref[0] + degp_ref[1] + 1.0
    dinv = lax.rsqrt(deg)
    dinv_ref[...] = dinv
    f0, f1, f2 = ft_ref[0], ft_ref[1], ft_ref[2]
    t0_ref[...] = (f0 * w_ref[0, 0] + f1 * w_ref[1, 0]
                   + f2 * w_ref[2, 0]) * dinv
    t1_ref[...] = (f0 * w_ref[0, 1] + f1 * w_ref[1, 1]
                   + f2 * w_ref[2, 1]) * dinv


def _tc_prep1(degp, ft, w_g1):
    shp = jax.ShapeDtypeStruct((NR, LANES), jnp.float32)
    return pl.pallas_call(
        _prep1_body,
        out_shape=[shp, shp, shp],
        in_specs=[
            pl.BlockSpec((2, NR, LANES), lambda: (0, 0, 0)),
            pl.BlockSpec((3, NR, LANES), lambda: (0, 0, 0)),
            pl.BlockSpec(memory_space=pltpu.SMEM),
        ],
        out_specs=[pl.BlockSpec((NR, LANES), lambda: (0, 0))] * 3,
    )(degp, ft, w_g1)


def _mid_body(acc1_ref, t0_ref, t1_ref, dinv_ref, w2_ref, b1_ref, t2_ref):
    dinv = dinv_ref[...]
    x0 = jnp.maximum(
        dinv * (acc1_ref[0, 0] + acc1_ref[1, 0] + t0_ref[...]) + b1_ref[0],
        0.0)
    x1 = jnp.maximum(
        dinv * (acc1_ref[0, 1] + acc1_ref[1, 1] + t1_ref[...]) + b1_ref[1],
        0.0)
    t2_ref[...] = (x0 * w2_ref[0, 0] + x1 * w2_ref[1, 0]) * dinv


def _tc_mid(acc1, t0, t1, dinv, w_g2, b_g1):
    blk = pl.BlockSpec((NR, LANES), lambda: (0, 0))
    return pl.pallas_call(
        _mid_body,
        out_shape=jax.ShapeDtypeStruct((NR, LANES), jnp.float32),
        in_specs=[
            pl.BlockSpec((2, 2, NR, LANES), lambda: (0, 0, 0, 0)),
            blk, blk, blk,
            pl.BlockSpec(memory_space=pltpu.SMEM),
            pl.BlockSpec(memory_space=pltpu.SMEM),
        ],
        out_specs=blk,
    )(acc1, t0, t1, dinv, w_g2, b_g1)


_HB = 7168
_HK = NP // _HB


def _head_body(a2_ref, t2_ref, dinv_ref, b2_ref, w1_ref, b1_ref,
               w2_ref, bb2_ref, w3_ref, b3_ref, out_ref, acc_ref):
    k = pl.program_id(0)
    x3 = jnp.maximum(
        dinv_ref[0:1] * (a2_ref[0:1] + a2_ref[1:2] + t2_ref[0:1])
        + b2_ref[0], 0.0)
    col = k * _HB + lax.broadcasted_iota(jnp.int32, (1, _HB), 1)
    prod = jnp.where(col < NN, w1_ref[...] * x3, 0.0)
    part = jnp.sum(prod, axis=1, keepdims=True)

    @pl.when(k == 0)
    def _():
        acc_ref[...] = jnp.zeros_like(acc_ref)
    acc_ref[...] += part

    @pl.when(k == _HK - 1)
    def _():
        y1 = jnp.maximum(acc_ref[...] + b1_ref[...], 0.0)
        y2 = jnp.maximum(
            jnp.dot(w2_ref[...], y1, preferred_element_type=jnp.float32,
                    precision=lax.Precision.HIGHEST) + bb2_ref[...], 0.0)
        out_ref[...] = jnp.dot(
            w3_ref[...], y2, preferred_element_type=jnp.float32,
            precision=lax.Precision.HIGHEST) + b3_ref[...]


def _tc_head(a2, t2r, dinvr, b_g2, w_fc1, b_fc1, w_fc2, b_fc2, w_fc, b_fc):
    row = pl.BlockSpec((1, _HB), lambda k: (0, k))
    return pl.pallas_call(
        _head_body,
        grid=(_HK,),
        out_shape=jax.ShapeDtypeStruct((10, 1), jnp.float32),
        in_specs=[
            pl.BlockSpec((2, _HB), lambda k: (0, k)),
            row, row,
            pl.BlockSpec(memory_space=pltpu.SMEM),
            pl.BlockSpec((128, _HB), lambda k: (0, k)),
            pl.BlockSpec((128, 1), lambda k: (0, 0)),
            pl.BlockSpec((128, 128), lambda k: (0, 0)),
            pl.BlockSpec((128, 1), lambda k: (0, 0)),
            pl.BlockSpec((10, 128), lambda k: (0, 0)),
            pl.BlockSpec((10, 1), lambda k: (0, 0)),
        ],
        out_specs=pl.BlockSpec((10, 1), lambda k: (0, 0)),
        scratch_shapes=[pltpu.VMEM((128, 1), jnp.float32)],
    )(a2, t2r, dinvr, b_g2, w_fc1, b_fc1, w_fc2, b_fc2, w_fc, b_fc)


def kernel(feature, edge_index, W_g1, b_g1, W_g2, b_g2,
           W_fc1, b_fc1, W_fc2, b_fc2, W_fc, b_fc):
    er = edge_index.reshape(2, ROWS, LANES)
    z = jnp.zeros((NP,), jnp.float32)
    ft = jnp.pad(feature.T, ((0, 0), (0, NP - NN)))
    ft = ft.reshape(3, NR, LANES)

    degp = _sc_degree(er, z)
    dinv, t0, t1 = _tc_prep1(degp.reshape(2, NR, LANES), ft, W_g1)

    acc1 = _sc_conv2ch(er, t0.reshape(NP), t1.reshape(NP), z)
    t2 = _tc_mid(acc1.reshape(NC, 2, NR, LANES), t0, t1, dinv, W_g2, b_g1)

    acc2 = _sc_conv1ch(er, t2.reshape(NP), z)

    out = _tc_head(
        acc2.reshape(2, NP), t2.reshape(1, NP), dinv.reshape(1, NP),
        b_g2, W_fc1, b_fc1.reshape(128, 1), W_fc2, b_fc2.reshape(128, 1),
        W_fc, b_fc.reshape(10, 1))
    return out.reshape(10)


# speedup vs baseline: 2.1259x; 2.1259x over previous
import functools

import jax
import jax.numpy as jnp
from jax import lax
from jax.experimental import pallas as pl
from jax.experimental.pallas import tpu as pltpu
from jax.experimental.pallas import tpu_sc as plsc

NN = 100000
EE = 6400000
LANES = 128
ROWS = EE // LANES
NP = 100352
NR = NP // LANES
NC, NS = 2, 16
NW = NC * NS
SL = NP // NS

_MESH = plsc.VectorSubcoreMesh(
    core_axis_name="c", subcore_axis_name="s", num_cores=NC, num_subcores=NS)


def _sc_degree_body(er, z, out, dst_v, ones_v, acc_sh, lsem, ssem):
    K = 16
    chunks = ROWS // K
    rem = chunks % NW
    cid = lax.axis_index("c")
    sid = lax.axis_index("s")
    wid = sid * NC + cid

    @pl.when(sid == 0)
    def _():
        pltpu.sync_copy(z, acc_sh)
    for i in range(8):
        ones_v[pl.ds(16 * i, 16)] = jnp.ones((16,), jnp.float32)
    plsc.subcore_barrier()

    n_i = jnp.where(wid < rem, chunks // NW + 1, chunks // NW)
    pltpu.async_copy(er.at[1, pl.ds(wid * K, K)], dst_v.at[0], lsem)

    def drain_scatters(n):
        for _ in range(n):
            pltpu.make_async_copy(z.at[pl.ds(0, LANES)], ones_v,
                                  ssem).wait()

    def body(i, carry):
        b3 = lax.rem(i, 3)
        pltpu.make_async_copy(er.at[1, pl.ds(0, K)], dst_v.at[0],
                              lsem).wait()

        @pl.when(i >= 2)
        def _():
            drain_scatters(K)

        @pl.when(i + 1 < n_i)
        def _():
            qn = wid + (i + 1) * NW
            pltpu.async_copy(er.at[1, pl.ds(qn * K, K)],
                             dst_v.at[lax.rem(i + 1, 3)], lsem)
        for j in range(K):
            pltpu.async_copy(ones_v, acc_sh.at[dst_v.at[b3, j]], ssem,
                             add=True)
        return carry

    lax.fori_loop(0, n_i, body, 0)
    drain_scatters(2 * K)
    plsc.subcore_barrier()
    pltpu.sync_copy(acc_sh.at[pl.ds(sid * SL, SL)],
                    out.at[cid, pl.ds(sid * SL, SL)])


_sc_degree = functools.partial(
    pl.kernel,
    out_type=jax.ShapeDtypeStruct((NC, NP), jnp.float32),
    mesh=_MESH,
    scratch_types=[
        pltpu.VMEM((3, 16, LANES), jnp.int32),
        pltpu.VMEM((LANES,), jnp.float32),
        pltpu.VMEM_SHARED((NP,), jnp.float32),
        pltpu.SemaphoreType.DMA,
        pltpu.SemaphoreType.DMA,
    ],
)(_sc_degree_body)


def _make_sc_conv(nch, K):
    chunks = ROWS // K
    rem = chunks % NW

    def body(er, *refs):
        tabs_hbm = refs[:nch]
        z = refs[nch]
        out = refs[nch + 1]
        src_v, dst_v = refs[nch + 2], refs[nch + 3]
        vals = refs[nch + 4:nch + 4 + nch]
        tabs_sh = refs[nch + 4 + nch:nch + 4 + 2 * nch]
        accs_sh = refs[nch + 4 + 2 * nch:nch + 4 + 3 * nch]
        lsem, gsem, ssem = refs[nch + 4 + 3 * nch:nch + 7 + 3 * nch]

        cid = lax.axis_index("c")
        sid = lax.axis_index("s")
        wid = sid * NC + cid

        @pl.when(sid == 0)
        def _():
            for ch in range(nch):
                pltpu.sync_copy(tabs_hbm[ch], tabs_sh[ch])
                pltpu.sync_copy(z, accs_sh[ch])
        plsc.subcore_barrier()

        n_i = jnp.where(wid < rem, chunks // NW + 1, chunks // NW)
        pltpu.async_copy(er.at[0, pl.ds(wid * K, K)], src_v.at[0], lsem)
        pltpu.async_copy(er.at[1, pl.ds(wid * K, K)], dst_v.at[0], lsem)

        def drain_scatters(n):
            for _ in range(n):
                pltpu.make_async_copy(z.at[pl.ds(0, LANES)],
                                      vals[0].at[0, 0], ssem).wait()

        def loop(i, carry):
            b2 = lax.rem(i, 2)
            b3 = lax.rem(i, 3)
            pltpu.make_async_copy(er.at[0, pl.ds(0, K)], src_v.at[0],
                                  lsem).wait()
            pltpu.make_async_copy(er.at[1, pl.ds(0, K)], dst_v.at[0],
                                  lsem).wait()

            @pl.when(i >= 2)
            def _():
                drain_scatters(K * nch)

            @pl.when(i + 1 < n_i)
            def _():
                qn = wid + (i + 1) * NW
                nb = lax.rem(i + 1, 3)
                pltpu.async_copy(er.at[0, pl.ds(qn * K, K)],
                                 src_v.at[nb], lsem)
                pltpu.async_copy(er.at[1, pl.ds(qn * K, K)],
                                 dst_v.at[nb], lsem)

            gds = [pltpu.async_copy(tabs_sh[ch].at[src_v.at[b3, j]],
                                    vals[ch].at[b2, j], gsem)
                   for j in range(K) for ch in range(nch)]
            for j in range(K):
                for ch in range(nch):
                    gds[j * nch + ch].wait()
                for ch in range(nch):
                    pltpu.async_copy(vals[ch].at[b2, j],
                                     accs_sh[ch].at[dst_v.at[b3, j]],
                                     ssem, add=True)
            return carry

        lax.fori_loop(0, n_i, loop, 0)
        drain_scatters(2 * K * nch)
        plsc.subcore_barrier()
        for ch in range(nch):
            pltpu.sync_copy(accs_sh[ch].at[pl.ds(sid * SL, SL)],
                            out.at[cid, ch, pl.ds(sid * SL, SL)])

    scratch = [pltpu.VMEM((3, K, LANES), jnp.int32),
               pltpu.VMEM((3, K, LANES), jnp.int32)]
    scratch += [pltpu.VMEM((2, K, LANES), jnp.float32) for _ in range(nch)]
    scratch += [pltpu.VMEM_SHARED((NP,), jnp.float32) for _ in range(2 * nch)]
    scratch += [pltpu.SemaphoreType.DMA, pltpu.SemaphoreType.DMA,
                pltpu.SemaphoreType.DMA]
    return functools.partial(
        pl.kernel,
        out_type=jax.ShapeDtypeStruct((NC, nch, NP), jnp.float32),
        mesh=_MESH,
        scratch_types=scratch,
    )(body)


_sc_conv2ch = _make_sc_conv(2, 8)
_sc_conv1ch = _make_sc_conv(1, 16)


def _prep1_body(degp_ref, ft_ref, w_ref, dinv_ref, t0_ref, t1_ref):
    deg = degp_ref[0] + degp_ref[1] + 1.0
    dinv = lax.rsqrt(deg)
    dinv_ref[...] = dinv
    f0, f1, f2 = ft_ref[0], ft_ref[1], ft_ref[2]
    t0_ref[...] = (f0 * w_ref[0, 0] + f1 * w_ref[1, 0]
                   + f2 * w_ref[2, 0]) * dinv
    t1_ref[...] = (f0 * w_ref[0, 1] + f1 * w_ref[1, 1]
                   + f2 * w_ref[2, 1]) * dinv


def _tc_prep1(degp, ft, w_g1):
    shp = jax.ShapeDtypeStruct((NR, LANES), jnp.float32)
    return pl.pallas_call(
        _prep1_body,
        out_shape=[shp, shp, shp],
        in_specs=[
            pl.BlockSpec((2, NR, LANES), lambda: (0, 0, 0)),
            pl.BlockSpec((3, NR, LANES), lambda: (0, 0, 0)),
            pl.BlockSpec(memory_space=pltpu.SMEM),
        ],
        out_specs=[pl.BlockSpec((NR, LANES), lambda: (0, 0))] * 3,
    )(degp, ft, w_g1)


def _mid_body(acc1_ref, t0_ref, t1_ref, dinv_ref, w2_ref, b1_ref, t2_ref):
    dinv = dinv_ref[...]
    x0 = jnp.maximum(
        dinv * (acc1_ref[0, 0] + acc1_ref[1, 0] + t0_ref[...]) + b1_ref[0],
        0.0)
    x1 = jnp.maximum(
        dinv * (acc1_ref[0, 1] + acc1_ref[1, 1] + t1_ref[...]) + b1_ref[1],
        0.0)
    t2_ref[...] = (x0 * w2_ref[0, 0] + x1 * w2_ref[1, 0]) * dinv


def _tc_mid(acc1, t0, t1, dinv, w_g2, b_g1):
    blk = pl.BlockSpec((NR, LANES), lambda: (0, 0))
    return pl.pallas_call(
        _mid_body,
        out_shape=jax.ShapeDtypeStruct((NR, LANES), jnp.float32),
        in_specs=[
            pl.BlockSpec((2, 2, NR, LANES), lambda: (0, 0, 0, 0)),
            blk, blk, blk,
            pl.BlockSpec(memory_space=pltpu.SMEM),
            pl.BlockSpec(memory_space=pltpu.SMEM),
        ],
        out_specs=blk,
    )(acc1, t0, t1, dinv, w_g2, b_g1)


_HB = 7168
_HK = NP // _HB


def _head_body(a2_ref, t2_ref, dinv_ref, b2_ref, w1_ref, b1_ref,
               w2_ref, bb2_ref, w3_ref, b3_ref, out_ref, acc_ref):
    k = pl.program_id(0)
    x3 = jnp.maximum(
        dinv_ref[0:1] * (a2_ref[0:1] + a2_ref[1:2] + t2_ref[0:1])
        + b2_ref[0], 0.0)
    col = k * _HB + lax.broadcasted_iota(jnp.int32, (1, _HB), 1)
    prod = jnp.where(col < NN, w1_ref[...] * x3, 0.0)
    part = jnp.sum(prod, axis=1, keepdims=True)

    @pl.when(k == 0)
    def _():
        acc_ref[...] = jnp.zeros_like(acc_ref)
    acc_ref[...] += part

    @pl.when(k == _HK - 1)
    def _():
        y1 = jnp.maximum(acc_ref[...] + b1_ref[...], 0.0)
        y2 = jnp.maximum(
            jnp.dot(w2_ref[...], y1, preferred_element_type=jnp.float32,
                    precision=lax.Precision.HIGHEST) + bb2_ref[...], 0.0)
        out_ref[...] = jnp.dot(
            w3_ref[...], y2, preferred_element_type=jnp.float32,
            precision=lax.Precision.HIGHEST) + b3_ref[...]


def _tc_head(a2, t2r, dinvr, b_g2, w_fc1, b_fc1, w_fc2, b_fc2, w_fc, b_fc):
    row = pl.BlockSpec((1, _HB), lambda k: (0, k))
    return pl.pallas_call(
        _head_body,
        grid=(_HK,),
        out_shape=jax.ShapeDtypeStruct((10, 1), jnp.float32),
        in_specs=[
            pl.BlockSpec((2, _HB), lambda k: (0, k)),
            row, row,
            pl.BlockSpec(memory_space=pltpu.SMEM),
            pl.BlockSpec((128, _HB), lambda k: (0, k)),
            pl.BlockSpec((128, 1), lambda k: (0, 0)),
            pl.BlockSpec((128, 128), lambda k: (0, 0)),
            pl.BlockSpec((128, 1), lambda k: (0, 0)),
            pl.BlockSpec((10, 128), lambda k: (0, 0)),
            pl.BlockSpec((10, 1), lambda k: (0, 0)),
        ],
        out_specs=pl.BlockSpec((10, 1), lambda k: (0, 0)),
        scratch_shapes=[pltpu.VMEM((128, 1), jnp.float32)],
    )(a2, t2r, dinvr, b_g2, w_fc1, b_fc1, w_fc2, b_fc2, w_fc, b_fc)


def kernel(feature, edge_index, W_g1, b_g1, W_g2, b_g2,
           W_fc1, b_fc1, W_fc2, b_fc2, W_fc, b_fc):
    er = edge_index.reshape(2, ROWS, LANES)
    z = jnp.zeros((NP,), jnp.float32)
    ft = jnp.pad(feature.T, ((0, 0), (0, NP - NN)))
    ft = ft.reshape(3, NR, LANES)

    degp = _sc_degree(er, z)
    dinv, t0, t1 = _tc_prep1(degp.reshape(2, NR, LANES), ft, W_g1)

    acc1 = _sc_conv2ch(er, t0.reshape(NP), t1.reshape(NP), z)
    t2 = _tc_mid(acc1.reshape(NC, 2, NR, LANES), t0, t1, dinv, W_g2, b_g1)

    acc2 = _sc_conv1ch(er, t2.reshape(NP), z)

    out = _tc_head(
        acc2.reshape(2, NP), t2.reshape(1, NP), dinv.reshape(1, NP),
        b_g2, W_fc1, b_fc1.reshape(128, 1), W_fc2, b_fc2.reshape(128, 1),
        W_fc, b_fc.reshape(10, 1))
    return out.reshape(10)
